# Initial kernel scaffold; baseline (speedup 1.0000x reference)
import jax
import jax.numpy as jnp
from jax.experimental import pallas as pl


def kernel(x, lengths, table, W1, b1, W2, b2):
    raise NotImplementedError("write your pallas kernel here")


# speedup vs baseline: 7.5152x; 7.5152x over previous
import functools

import jax
import jax.numpy as jnp
from jax import lax
from jax.experimental import pallas as pl
from jax.experimental.pallas import tpu as pltpu
from jax.experimental.pallas import tpu_sc as plsc

_LANES = 16
_CH0 = 128


@functools.cache
def _make_pool(B, L, V, D):
    info = plsc.get_sparse_core_info()
    NW = info.num_cores * info.num_subcores
    bpw = B // NW
    C = D // _LANES
    CH1 = L - _CH0
    mesh = plsc.VectorSubcoreMesh(core_axis_name="c", subcore_axis_name="s")

    @functools.partial(
        pl.kernel,
        mesh=mesh,
        out_type=jax.ShapeDtypeStruct((B, D), jnp.float32),
        scratch_types=[
            pltpu.VMEM((bpw * L,), jnp.int32),
            pltpu.VMEM((L, D), jnp.float32),
            pltpu.VMEM((bpw, D), jnp.float32),
            pltpu.SemaphoreType.DMA,
        ],
    )
    def pool(x_hbm, table_hbm, out_hbm, idx_v, rows_v, out_v, sem):
        wid = lax.axis_index("s") * info.num_cores + lax.axis_index("c")
        base = wid * bpw
        pltpu.sync_copy(x_hbm.at[pl.ds(base * L, bpw * L)], idx_v)

        def sample_body(s, carry):
            off = pl.multiple_of(s * L, 8)
            h0 = pltpu.async_copy(
                table_hbm.at[idx_v.at[pl.ds(off, _CH0)]],
                rows_v.at[pl.ds(0, _CH0), :], sem)
            h1 = pltpu.async_copy(
                table_hbm.at[idx_v.at[pl.ds(off + _CH0, CH1)]],
                rows_v.at[pl.ds(_CH0, CH1), :], sem)
            h0.wait()
            h1.wait()

            UNROLL = 4
            def acc_body(r, accs):
                new = list(accs)
                for u in range(UNROLL):
                    for c in range(C):
                        new[c] = new[c] + rows_v[r * UNROLL + u,
                                                 pl.ds(c * _LANES, _LANES)]
                return tuple(new)

            zero = jnp.zeros((_LANES,), jnp.float32)
            accs = lax.fori_loop(0, L // UNROLL, acc_body, (zero,) * C)
            for r in range((L // UNROLL) * UNROLL, L):
                accs = tuple(accs[c] + rows_v[r, pl.ds(c * _LANES, _LANES)]
                             for c in range(C))
            for c in range(C):
                out_v[s, pl.ds(c * _LANES, _LANES)] = accs[c]
            return carry

        lax.fori_loop(0, bpw, sample_body, 0)
        pltpu.sync_copy(out_v, out_hbm.at[pl.ds(base, bpw), :])

    return pool


def _mlp_body(s_ref, l_ref, w1_ref, b1_ref, w2_ref, b2_ref, o_ref):
    inv = 1.0 / l_ref[...].astype(jnp.float32)
    rep = s_ref[...] * inv
    h = lax.dot_general(rep, w1_ref[...], (((1,), (1,)), ((), ())),
                        preferred_element_type=jnp.float32) + b1_ref[...]
    h = jnp.maximum(h, 0.0)
    o_ref[...] = lax.dot_general(h, w2_ref[...], (((1,), (1,)), ((), ())),
                                 preferred_element_type=jnp.float32) + b2_ref[...]


def _mlp(sums, len2d, W1, b1r, W2p, b2p):
    B, D = sums.shape
    BLK = 512
    return pl.pallas_call(
        _mlp_body,
        grid=(B // BLK,),
        in_specs=[
            pl.BlockSpec((BLK, D), lambda i: (i, 0)),
            pl.BlockSpec((BLK, 1), lambda i: (i, 0)),
            pl.BlockSpec((D, D), lambda i: (0, 0)),
            pl.BlockSpec((1, D), lambda i: (0, 0)),
            pl.BlockSpec((D, D), lambda i: (0, 0)),
            pl.BlockSpec((1, D), lambda i: (0, 0)),
        ],
        out_specs=pl.BlockSpec((BLK, D), lambda i: (i, 0)),
        out_shape=jax.ShapeDtypeStruct((B, D), jnp.float32),
    )(sums, len2d, W1, b1r, W2p, b2p)


def kernel(x, lengths, table, W1, b1, W2, b2):
    B, L = x.shape
    V, D = table.shape
    OUT = W2.shape[0]
    pool = _make_pool(B, L, V, D)
    sums = pool(x.reshape(-1), table)
    W2p = jnp.zeros((D, D), W2.dtype).at[:OUT].set(W2)
    b2p = jnp.zeros((1, D), b2.dtype).at[0, :OUT].set(b2)
    logits = _mlp(sums, lengths.reshape(B, 1), W1, b1.reshape(1, D), W2p, b2p)
    return logits[:, :OUT]


# speedup vs baseline: 13.0112x; 1.7313x over previous
import functools

import jax
import jax.numpy as jnp
from jax import lax
from jax.experimental import pallas as pl
from jax.experimental.pallas import tpu as pltpu
from jax.experimental.pallas import tpu_sc as plsc

_LANES = 16
_CH0 = 128


@functools.cache
def _make_pool(B, L, V, D):
    info = plsc.get_sparse_core_info()
    NW = info.num_cores * info.num_subcores
    bpw = B // NW
    C = D // _LANES
    CH1 = L - _CH0
    mesh = plsc.VectorSubcoreMesh(core_axis_name="c", subcore_axis_name="s")

    @functools.partial(
        pl.kernel,
        mesh=mesh,
        out_type=jax.ShapeDtypeStruct((B, D), jnp.float32),
        scratch_types=[
            pltpu.VMEM((bpw * L,), jnp.int32),
            pltpu.VMEM((2, L, D), jnp.float32),
            pltpu.VMEM((bpw, D), jnp.float32),
            pltpu.SemaphoreType.DMA,
        ],
    )
    def pool(x_hbm, table_hbm, out_hbm, idx_v, rows_v, out_v, sem):
        wid = lax.axis_index("s") * info.num_cores + lax.axis_index("c")
        base = wid * bpw
        pltpu.sync_copy(x_hbm.at[pl.ds(base * L, bpw * L)], idx_v)

        def copies(s, buf):
            off = pl.multiple_of(s * L, 8)
            return (
                pltpu.make_async_copy(
                    table_hbm.at[idx_v.at[pl.ds(off, _CH0)]],
                    rows_v.at[buf, pl.ds(0, _CH0), :], sem),
                pltpu.make_async_copy(
                    table_hbm.at[idx_v.at[pl.ds(off + _CH0, CH1)]],
                    rows_v.at[buf, pl.ds(_CH0, CH1), :], sem),
            )

        def issue(s, buf):
            for cp in copies(s, buf):
                cp.start()

        def wait(s, buf):
            for cp in copies(s, buf):
                cp.wait()

        def accumulate(s, buf):
            UNROLL = 4
            def acc_body(r, accs):
                new = list(accs)
                for u in range(UNROLL):
                    for c in range(C):
                        new[c] = new[c] + rows_v[buf, r * UNROLL + u,
                                                 pl.ds(c * _LANES, _LANES)]
                return tuple(new)

            zero = jnp.zeros((_LANES,), jnp.float32)
            accs = lax.fori_loop(0, L // UNROLL, acc_body, (zero,) * C)
            for r in range((L // UNROLL) * UNROLL, L):
                accs = tuple(accs[c] + rows_v[buf, r, pl.ds(c * _LANES, _LANES)]
                             for c in range(C))
            for c in range(C):
                out_v[s, pl.ds(c * _LANES, _LANES)] = accs[c]

        issue(0, 0)

        def pair_body(p, carry):
            s0 = 2 * p
            issue(s0 + 1, 1)
            wait(s0, 0)
            accumulate(s0, 0)

            @pl.when(s0 + 2 < bpw)
            def _():
                issue(s0 + 2, 0)

            wait(s0 + 1, 1)
            accumulate(s0 + 1, 1)
            return carry

        lax.fori_loop(0, bpw // 2, pair_body, 0)
        pltpu.sync_copy(out_v, out_hbm.at[pl.ds(base, bpw), :])

    return pool


def _mlp_body(s_ref, l_ref, w1_ref, b1_ref, w2_ref, b2_ref, o_ref):
    inv = 1.0 / l_ref[...].astype(jnp.float32)
    rep = s_ref[...] * inv
    h = lax.dot_general(rep, w1_ref[...], (((1,), (1,)), ((), ())),
                        preferred_element_type=jnp.float32) + b1_ref[...]
    h = jnp.maximum(h, 0.0)
    o_ref[...] = lax.dot_general(h, w2_ref[...], (((1,), (1,)), ((), ())),
                                 preferred_element_type=jnp.float32) + b2_ref[...]


def _mlp(sums, len2d, W1, b1r, W2p, b2p):
    B, D = sums.shape
    BLK = 512
    return pl.pallas_call(
        _mlp_body,
        grid=(B // BLK,),
        in_specs=[
            pl.BlockSpec((BLK, D), lambda i: (i, 0)),
            pl.BlockSpec((BLK, 1), lambda i: (i, 0)),
            pl.BlockSpec((D, D), lambda i: (0, 0)),
            pl.BlockSpec((1, D), lambda i: (0, 0)),
            pl.BlockSpec((D, D), lambda i: (0, 0)),
            pl.BlockSpec((1, D), lambda i: (0, 0)),
        ],
        out_specs=pl.BlockSpec((BLK, D), lambda i: (i, 0)),
        out_shape=jax.ShapeDtypeStruct((B, D), jnp.float32),
    )(sums, len2d, W1, b1r, W2p, b2p)


def kernel(x, lengths, table, W1, b1, W2, b2):
    B, L = x.shape
    V, D = table.shape
    OUT = W2.shape[0]
    pool = _make_pool(B, L, V, D)
    sums = pool(x.reshape(-1), table)
    W2p = jnp.zeros((D, D), W2.dtype).at[:OUT].set(W2)
    b2p = jnp.zeros((1, D), b2.dtype).at[0, :OUT].set(b2)
    logits = _mlp(sums, lengths.reshape(B, 1), W1, b1.reshape(1, D), W2p, b2p)
    return logits[:, :OUT]
